# baseline (speedup 1.0000x reference)
import functools

import jax
import jax.numpy as jnp
from jax import lax
from jax.experimental import pallas as pl
from jax.experimental.pallas import tpu as pltpu
from jax.experimental.pallas import tpu_sc as plsc

H = 768
R = 768
KN = 100000
B = 32
TOPK = 100

KB = 2048
NSTEP = -(-KN // KB)
KPAD = NSTEP * KB
PADW = 128
NV = KN // 16
NBINS = 1024
UNROLL = 10
HROW = 1040
HSTRIDE = HROW + 1


def _ln(x, g, b):
    m = x.mean(-1, keepdims=True)
    d = x - m
    v = (d * d).mean(-1, keepdims=True)
    return d * jax.lax.rsqrt(v + 1e-5) * g + b


def _encode_score_body(query_ref, wq_ref, wk_ref, vec_ref, know_ref,
                       out_ref, q_s):
    @pl.when(pl.program_id(0) == 0)
    def _():
        x = jnp.dot(query_ref[...], wq_ref[...],
                    preferred_element_type=jnp.float32) + vec_ref[0:1, :]
        q_s[...] = _ln(x, vec_ref[1:2, :], vec_ref[2:3, :])

    kb = jnp.dot(know_ref[...].astype(jnp.bfloat16),
                 wk_ref[...].astype(jnp.bfloat16),
                 preferred_element_type=jnp.float32) + vec_ref[3:4, :]
    kenc = _ln(kb, vec_ref[4:5, :], vec_ref[5:6, :])
    sb = jax.lax.dot_general(
        q_s[...].astype(jnp.bfloat16), kenc.astype(jnp.bfloat16),
        (((1,), (1,)), ((), ())),
        preferred_element_type=jnp.float32)
    v = jax.lax.bitcast_convert_type(sb, jnp.int32)
    out_ref[...] = jnp.where(v >= 0, v, v ^ jnp.int32(0x7FFFFFFF))


def _encode_score(q2, wq, wk, vecs, knowledge):
    return pl.pallas_call(
        _encode_score_body,
        grid=(NSTEP,),
        in_specs=[
            pl.BlockSpec((B, H), lambda i: (0, 0)),
            pl.BlockSpec((H, R), lambda i: (0, 0)),
            pl.BlockSpec((R, R), lambda i: (0, 0)),
            pl.BlockSpec((8, R), lambda i: (0, 0)),
            pl.BlockSpec((KB, R), lambda i: (i, 0)),
        ],
        out_specs=pl.BlockSpec((B, KB), lambda i: (0, i)),
        out_shape=jax.ShapeDtypeStruct((B, KPAD), jnp.int32),
        scratch_shapes=[pltpu.VMEM((B, R), jnp.float32)],
    )(q2, wq, wk, vecs, knowledge)


def _topk_kernel(scores_hbm, topv_hbm, topi_hbm,
                 data, hist, mh, sfx, out_v, out_i, st):
    wid = lax.axis_index("s") * 2 + lax.axis_index("c")
    lane = lax.iota(jnp.int32, 16)
    lane_off = lane * HSTRIDE
    ones = jnp.ones((16,), jnp.int32)
    zero16 = jnp.zeros((16,), jnp.int32)

    pltpu.sync_copy(scores_hbm.at[pl.ds(wid * KPAD, KN)], data)

    for j in range(PADW // 16):
        out_v[pl.ds(j * 16, 16)] = zero16
        out_i[pl.ds(j * 16, 16)] = wid * PADW + j * 16 + lane

    def _zero_hist():
        def zb(g, c):
            for u in range(8):
                hist[pl.ds((g * 8 + u) * 16, 16)] = zero16
            return c
        lax.fori_loop(0, (16 * HROW) // 128, zb, 0)

    def _merge_and_select(k_need):
        def mb(g, c):
            acc = zero16
            for l in range(16):
                acc = acc + hist[pl.ds(l * HSTRIDE + g * 16, 16)]
            mh[pl.ds(g * 16, 16)] = acc
            return c
        lax.fori_loop(0, NBINS // 16, mb, 0)

        def sb(gg, carry):
            g = NBINS // 16 - 1 - gg
            v = mh[pl.ds(g * 16, 16)]
            c = lax.cumsum(lax.rev(v, (0,)), axis=0) + carry
            sfx[pl.ds(g * 16, 16)] = lax.rev(c, (0,))
            return carry + jnp.sum(v)
        lax.fori_loop(0, NBINS // 16, sb, jnp.int32(0))

        def cb(g, c):
            s = sfx[pl.ds(g * 16, 16)]
            return c + jnp.sum(jnp.where(s >= k_need, 1, 0).astype(jnp.int32))
        cnt = lax.fori_loop(0, NBINS // 16, cb, jnp.int32(0))
        bstar = jnp.maximum(cnt - 1, 0)
        bvec = jnp.full((16,), bstar, jnp.int32)
        s_b = jnp.max(plsc.load_gather(sfx, [bvec]))
        cntb = jnp.max(plsc.load_gather(mh, [bvec]))
        return bstar, s_b, cntb

    _zero_hist()

    def l0(g, c):
        base = g * UNROLL
        keys = [data[pl.ds((base + u) * 16, 16)] for u in range(UNROLL)]
        idxs = [((k >> 22) + 512) + lane_off for k in keys]
        for ix in idxs:
            plsc.addupdate_scatter(hist, [ix], ones)
        return c
    lax.fori_loop(0, NV // UNROLL, l0, 0)

    bstar, s_b, cntb = _merge_and_select(jnp.int32(TOPK))
    k1 = jnp.int32(TOPK) - (s_b - cntb)
    st[0] = bstar - 512
    st[1] = k1
    st[2] = jnp.int32(22)
    st[3] = jnp.where(cntb == k1, 1, 0).astype(jnp.int32)

    for (sh, nbits, bmask) in ((12, 10, 1023), (2, 10, 1023), (0, 2, 3)):
        @pl.when(st[3] == 0)
        def _(sh=sh, nbits=nbits, bmask=bmask):
            pref = st[0]
            k_need = st[1]
            _zero_hist()

            def lb(g, c):
                base = g * UNROLL
                keys = [data[pl.ds((base + u) * 16, 16)]
                        for u in range(UNROLL)]
                ms = [(k >> (sh + nbits)) == pref for k in keys]
                while len(ms) > 1:
                    ms = [jnp.logical_or(ms[i], ms[i + 1])
                          for i in range(0, len(ms) - 1, 2)] + (
                              [ms[-1]] if len(ms) % 2 else [])
                hits = jnp.max(plsc.all_reduce_population_count(ms[0]))

                @pl.when(hits > 0)
                def _():
                    keys2 = [data[pl.ds((base + u) * 16, 16)]
                             for u in range(UNROLL)]
                    msks = [(k >> (sh + nbits)) == pref for k in keys2]
                    bbs = [((k >> sh) & bmask) + lane_off for k in keys2]
                    for bb, msk in zip(bbs, msks):
                        plsc.addupdate_scatter(hist, [bb], ones, mask=msk)
                return c
            lax.fori_loop(0, NV // UNROLL, lb, 0)

            bs, sbv, cbv = _merge_and_select(k_need)
            newk = k_need - (sbv - cbv)
            st[0] = (st[0] << nbits) | bs
            st[1] = newk
            st[2] = jnp.int32(sh)
            st[3] = jnp.where(cbv == newk, 1, 0).astype(jnp.int32)

    sf = st[2]
    pref = st[0]
    r_final = st[1]
    shv = jnp.full((16,), sf, jnp.int32)
    st[4] = jnp.int32(0)
    st[5] = jnp.int32(0)

    def ex(g, c):
        base = g * UNROLL
        keys = [data[pl.ds((base + u) * 16, 16)] for u in range(UNROLL)]
        ms = [lax.shift_right_arithmetic(k, shv) >= pref for k in keys]
        while len(ms) > 1:
            ms = [jnp.logical_or(ms[i], ms[i + 1])
                  for i in range(0, len(ms) - 1, 2)] + (
                      [ms[-1]] if len(ms) % 2 else [])
        hits = jnp.max(plsc.all_reduce_population_count(ms[0]))

        @pl.when(hits > 0)
        def _():
            for u in range(UNROLL):
                i = g * UNROLL + u
                key = data[pl.ds(i * 16, 16)]
                pf = lax.shift_right_arithmetic(key, shv)
                m_ge = pf >= pref
                nge = jnp.max(plsc.all_reduce_population_count(m_ge))

                @pl.when(nge > 0)
                def _(i=i, key=key, pf=pf):
                    optr = st[4]
                    tptr = st[5]
                    m_def = pf > pref
                    m_tie = pf == pref
                    ic_t = lax.cumsum(
                        jnp.where(m_tie, 1, 0).astype(jnp.int32), axis=0)
                    m_tie2 = jnp.logical_and(m_tie,
                                             (tptr + ic_t - 1) < r_final)
                    m = jnp.logical_or(m_def, m_tie2)
                    ic = lax.cumsum(
                        jnp.where(m, 1, 0).astype(jnp.int32), axis=0)
                    rank = optr + ic - 1
                    plsc.store_scatter(out_v, [rank], key, mask=m)
                    plsc.store_scatter(out_i, [rank], i * 16 + lane, mask=m)
                    st[4] = optr + jnp.max(ic)
                    st[5] = tptr + jnp.max(ic_t)
        return c
    lax.fori_loop(0, NV // UNROLL, ex, 0)

    pltpu.sync_copy(out_v, topv_hbm.at[pl.ds(wid * PADW, PADW)])
    pltpu.sync_copy(out_i, topi_hbm.at[pl.ds(wid * PADW, PADW)])


def _topk(scores_flat):
    mesh = plsc.VectorSubcoreMesh(core_axis_name="c", subcore_axis_name="s")
    return pl.kernel(
        _topk_kernel,
        out_type=(jax.ShapeDtypeStruct((B * PADW,), jnp.int32),
                  jax.ShapeDtypeStruct((B * PADW,), jnp.int32)),
        mesh=mesh,
        scratch_types=[
            pltpu.VMEM((KN,), jnp.int32),
            pltpu.VMEM((16 * HROW,), jnp.int32),
            pltpu.VMEM((NBINS,), jnp.int32),
            pltpu.VMEM((NBINS,), jnp.int32),
            pltpu.VMEM((PADW,), jnp.int32),
            pltpu.VMEM((PADW,), jnp.int32),
            pltpu.SMEM((8,), jnp.int32),
        ],
        compiler_params=pltpu.CompilerParams(needs_layout_passes=False),
    )(scores_flat)


def _gather_kernel(topi_hbm, know_hbm, out_hbm, idx_v, rows_v, sem):
    wid = lax.axis_index("s") * 2 + lax.axis_index("c")
    pltpu.sync_copy(topi_hbm.at[pl.ds(wid * PADW, PADW)], idx_v)
    pltpu.async_copy(know_hbm.at[idx_v], rows_v, sem).wait()
    pltpu.sync_copy(rows_v, out_hbm.at[pl.ds(wid * PADW, PADW)])


def _gather_rows(topi_flat, knowledge):
    mesh = plsc.VectorSubcoreMesh(core_axis_name="c", subcore_axis_name="s")
    return pl.kernel(
        _gather_kernel,
        out_type=jax.ShapeDtypeStruct((B * PADW, R), jnp.float32),
        mesh=mesh,
        scratch_types=[
            pltpu.VMEM((PADW,), jnp.int32),
            pltpu.VMEM((PADW, R), jnp.float32),
            pltpu.SemaphoreType.DMA,
        ],
        compiler_params=pltpu.CompilerParams(needs_layout_passes=False),
    )(topi_flat, knowledge)


def _finish_body(gath_ref, wq_ref, query_ref, wk_ref, wo_ref,
                 wf1_ref, wf2_ref, vec_ref, out_ref):
    kb = jnp.dot(gath_ref[...], wk_ref[...],
                 preferred_element_type=jnp.float32) + vec_ref[0:1, :]
    kenc = _ln(kb, vec_ref[1:2, :], vec_ref[2:3, :])

    q2 = _ln(jnp.dot(query_ref[...], wq_ref[...],
                     preferred_element_type=jnp.float32) + vec_ref[10:11, :],
             vec_ref[11:12, :], vec_ref[12:13, :])
    sall = jax.lax.dot_general(
        q2, kenc, (((1,), (1,)), ((), ())),
        preferred_element_type=jnp.float32)
    tv = jnp.concatenate(
        [sall[b:b + 1, b * PADW:(b + 1) * PADW] for b in range(B)], axis=0)
    col = lax.broadcasted_iota(jnp.int32, (B, PADW), 1)
    tvm = jnp.where(col < TOPK, tv, -jnp.inf)
    mx = jnp.max(tvm, axis=-1, keepdims=True)
    e = jnp.exp(tvm - mx)
    w = e / jnp.sum(e, axis=-1, keepdims=True)

    wt = jnp.tile(w, (1, B))
    colb = lax.broadcasted_iota(jnp.int32, (B, B * PADW), 1) // PADW
    rowb = lax.broadcasted_iota(jnp.int32, (B, B * PADW), 0)
    w2 = jnp.where(colb == rowb, wt, 0.0)
    ctx = jnp.dot(w2, kenc, preferred_element_type=jnp.float32)

    out = jnp.dot(ctx, wo_ref[...],
                  preferred_element_type=jnp.float32) + vec_ref[3:4, :]
    q = query_ref[...]
    h1 = (jnp.dot(q, wf1_ref[0:H, :], preferred_element_type=jnp.float32)
          + jnp.dot(out, wf1_ref[H:2 * H, :],
                    preferred_element_type=jnp.float32)
          + vec_ref[4:5, :])
    h = jax.nn.gelu(_ln(h1, vec_ref[5:6, :], vec_ref[6:7, :]))
    gate = jax.nn.sigmoid(
        jnp.dot(h, wf2_ref[...], preferred_element_type=jnp.float32)
        + vec_ref[7:8, :])
    out_ref[...] = _ln(q + gate * out, vec_ref[8:9, :], vec_ref[9:10, :])


def _finish(gathered, wq, q2, wk, wo, wf1, wf2, vecs):
    return pl.pallas_call(
        _finish_body,
        in_specs=[
            pl.BlockSpec((B * PADW, R), lambda: (0, 0)),
            pl.BlockSpec((H, R), lambda: (0, 0)),
            pl.BlockSpec((B, H), lambda: (0, 0)),
            pl.BlockSpec((R, R), lambda: (0, 0)),
            pl.BlockSpec((R, H), lambda: (0, 0)),
            pl.BlockSpec((2 * H, H), lambda: (0, 0)),
            pl.BlockSpec((H, H), lambda: (0, 0)),
            pl.BlockSpec((16, H), lambda: (0, 0)),
        ],
        out_specs=pl.BlockSpec((B, H), lambda: (0, 0)),
        out_shape=jax.ShapeDtypeStruct((B, H), jnp.float32),
    )(gathered, wq, q2, wk, wo, wf1, wf2, vecs)


def kernel(query_states, knowledge_embeddings, params):
    p = params
    q2 = query_states.reshape(B, H)
    z = jnp.zeros((H,), jnp.float32)
    vecs1 = jnp.stack([p['bq'], p['gq'], p['betaq'],
                       p['bk'], p['gk'], p['betak'], z, z])
    scores = _encode_score(q2, p['Wq'], p['Wk'], vecs1, knowledge_embeddings)
    topv_flat, topi_flat = _topk(scores.reshape(-1))
    gathered = _gather_rows(topi_flat, knowledge_embeddings)
    vecs2 = jnp.stack([p['bk'], p['gk'], p['betak'], p['bo'], p['bf1'],
                       p['gf'], p['betaf'], p['bf2'], p['gln'], p['bln'],
                       p['bq'], p['gq'], p['betaq'], z, z, z])
    fused = _finish(gathered, p['Wq'], q2,
                    p['Wk'], p['Wo'], p['Wf1'], p['Wf2'], vecs2)
    return fused.reshape(B, 1, H)


# speedup vs baseline: 1.0049x; 1.0049x over previous
import jax
import jax.numpy as jnp
from jax import lax
from jax.experimental import pallas as pl
from jax.experimental.pallas import tpu as pltpu
from jax.experimental.pallas import tpu_sc as plsc

H = 768
R = 768
KN = 100000
B = 32
TOPK = 100

KB = 4096
NSTEP = -(-KN // KB)
KPAD = NSTEP * KB
PADW = 128
NV = KN // 16
NBINS = 1024
UNROLL = 10
HROW = 1040
HSTRIDE = HROW + 1


def _ln(x, g, b):
    m = x.mean(-1, keepdims=True)
    d = x - m
    v = (d * d).mean(-1, keepdims=True)
    return d * jax.lax.rsqrt(v + 1e-5) * g + b


def _encode_score_body(query_ref, wq_ref, wk_ref, vec_ref, know_ref,
                       out_ref, q_s):
    @pl.when(pl.program_id(0) == 0)
    def _():
        x = jnp.dot(query_ref[...], wq_ref[...],
                    preferred_element_type=jnp.float32) + vec_ref[0:1, :]
        q_s[...] = _ln(x, vec_ref[1:2, :], vec_ref[2:3, :])

    kb = jnp.dot(know_ref[...], wk_ref[...],
                 preferred_element_type=jnp.float32) + vec_ref[3:4, :]
    kenc = _ln(kb, vec_ref[4:5, :], vec_ref[5:6, :])
    sb = jax.lax.dot_general(
        q_s[...], kenc, (((1,), (1,)), ((), ())),
        preferred_element_type=jnp.float32)
    v = jax.lax.bitcast_convert_type(sb, jnp.int32)
    out_ref[...] = jnp.where(v >= 0, v, v ^ jnp.int32(0x7FFFFFFF))


def _encode_score(q2, wq, wk, vecs, knowledge):
    return pl.pallas_call(
        _encode_score_body,
        grid=(NSTEP,),
        in_specs=[
            pl.BlockSpec((B, H), lambda i: (0, 0)),
            pl.BlockSpec((H, R), lambda i: (0, 0)),
            pl.BlockSpec((R, R), lambda i: (0, 0)),
            pl.BlockSpec((8, R), lambda i: (0, 0)),
            pl.BlockSpec((KB, R), lambda i: (i, 0)),
        ],
        out_specs=pl.BlockSpec((B, KB), lambda i: (0, i)),
        out_shape=jax.ShapeDtypeStruct((B, KPAD), jnp.int32),
        scratch_shapes=[pltpu.VMEM((B, R), jnp.float32)],
    )(q2, wq, wk, vecs, knowledge)


def _topk_kernel(scores_hbm, topv_hbm, topi_hbm,
                 data, hist, mh, sfx, out_v, out_i, st):
    wid = lax.axis_index("s") * 2 + lax.axis_index("c")
    lane = lax.iota(jnp.int32, 16)
    lane_off = lane * HSTRIDE
    ones = jnp.ones((16,), jnp.int32)
    zero16 = jnp.zeros((16,), jnp.int32)

    pltpu.sync_copy(scores_hbm.at[pl.ds(wid * KPAD, KN)], data)

    for j in range(PADW // 16):
        out_v[pl.ds(j * 16, 16)] = zero16
        out_i[pl.ds(j * 16, 16)] = wid * PADW + j * 16 + lane

    def _zero_hist():
        def zb(g, c):
            for u in range(8):
                hist[pl.ds((g * 8 + u) * 16, 16)] = zero16
            return c
        lax.fori_loop(0, (16 * HROW) // 128, zb, 0)

    def _merge_and_select(k_need):
        def mb(g, c):
            acc = zero16
            for l in range(16):
                acc = acc + hist[pl.ds(l * HSTRIDE + g * 16, 16)]
            mh[pl.ds(g * 16, 16)] = acc
            return c
        lax.fori_loop(0, NBINS // 16, mb, 0)

        def sb(gg, carry):
            g = NBINS // 16 - 1 - gg
            v = mh[pl.ds(g * 16, 16)]
            c = lax.cumsum(lax.rev(v, (0,)), axis=0) + carry
            sfx[pl.ds(g * 16, 16)] = lax.rev(c, (0,))
            return carry + jnp.sum(v)
        lax.fori_loop(0, NBINS // 16, sb, jnp.int32(0))

        def cb(g, c):
            s = sfx[pl.ds(g * 16, 16)]
            return c + jnp.sum(jnp.where(s >= k_need, 1, 0).astype(jnp.int32))
        cnt = lax.fori_loop(0, NBINS // 16, cb, jnp.int32(0))
        bstar = jnp.maximum(cnt - 1, 0)
        bvec = jnp.full((16,), bstar, jnp.int32)
        s_b = jnp.max(plsc.load_gather(sfx, [bvec]))
        cntb = jnp.max(plsc.load_gather(mh, [bvec]))
        return bstar, s_b, cntb

    _zero_hist()

    def l0(g, c):
        base = g * UNROLL
        keys = [data[pl.ds((base + u) * 16, 16)] for u in range(UNROLL)]
        idxs = [((k >> 22) + 512) + lane_off for k in keys]
        for ix in idxs:
            plsc.addupdate_scatter(hist, [ix], ones)
        return c
    lax.fori_loop(0, NV // UNROLL, l0, 0)

    bstar, s_b, cntb = _merge_and_select(jnp.int32(TOPK))
    k1 = jnp.int32(TOPK) - (s_b - cntb)
    st[0] = bstar - 512
    st[1] = k1
    st[2] = jnp.int32(22)
    st[3] = jnp.where(cntb == k1, 1, 0).astype(jnp.int32)

    for (sh, nbits, bmask) in ((12, 10, 1023), (2, 10, 1023), (0, 2, 3)):
        @pl.when(st[3] == 0)
        def _(sh=sh, nbits=nbits, bmask=bmask):
            pref = st[0]
            k_need = st[1]
            _zero_hist()

            def lb(g, c):
                base = g * UNROLL
                keys = [data[pl.ds((base + u) * 16, 16)]
                        for u in range(UNROLL)]
                ms = [(k >> (sh + nbits)) == pref for k in keys]
                while len(ms) > 1:
                    ms = [jnp.logical_or(ms[i], ms[i + 1])
                          for i in range(0, len(ms) - 1, 2)] + (
                              [ms[-1]] if len(ms) % 2 else [])
                hits = jnp.max(plsc.all_reduce_population_count(ms[0]))

                @pl.when(hits > 0)
                def _():
                    keys2 = [data[pl.ds((base + u) * 16, 16)]
                             for u in range(UNROLL)]
                    msks = [(k >> (sh + nbits)) == pref for k in keys2]
                    bbs = [((k >> sh) & bmask) + lane_off for k in keys2]
                    for bb, msk in zip(bbs, msks):
                        plsc.addupdate_scatter(hist, [bb], ones, mask=msk)
                return c
            lax.fori_loop(0, NV // UNROLL, lb, 0)

            bs, sbv, cbv = _merge_and_select(k_need)
            newk = k_need - (sbv - cbv)
            st[0] = (st[0] << nbits) | bs
            st[1] = newk
            st[2] = jnp.int32(sh)
            st[3] = jnp.where(cbv == newk, 1, 0).astype(jnp.int32)

    sf = st[2]
    pref = st[0]
    r_final = st[1]
    shv = jnp.full((16,), sf, jnp.int32)
    st[4] = jnp.int32(0)
    st[5] = jnp.int32(0)

    def ex(g, c):
        base = g * UNROLL
        keys = [data[pl.ds((base + u) * 16, 16)] for u in range(UNROLL)]
        ms = [lax.shift_right_arithmetic(k, shv) >= pref for k in keys]
        while len(ms) > 1:
            ms = [jnp.logical_or(ms[i], ms[i + 1])
                  for i in range(0, len(ms) - 1, 2)] + (
                      [ms[-1]] if len(ms) % 2 else [])
        hits = jnp.max(plsc.all_reduce_population_count(ms[0]))

        @pl.when(hits > 0)
        def _():
            for u in range(UNROLL):
                i = g * UNROLL + u
                key = data[pl.ds(i * 16, 16)]
                pf = lax.shift_right_arithmetic(key, shv)
                m_ge = pf >= pref
                nge = jnp.max(plsc.all_reduce_population_count(m_ge))

                @pl.when(nge > 0)
                def _(i=i, key=key, pf=pf):
                    optr = st[4]
                    tptr = st[5]
                    m_def = pf > pref
                    m_tie = pf == pref
                    ic_t = lax.cumsum(
                        jnp.where(m_tie, 1, 0).astype(jnp.int32), axis=0)
                    m_tie2 = jnp.logical_and(m_tie,
                                             (tptr + ic_t - 1) < r_final)
                    m = jnp.logical_or(m_def, m_tie2)
                    ic = lax.cumsum(
                        jnp.where(m, 1, 0).astype(jnp.int32), axis=0)
                    rank = optr + ic - 1
                    plsc.store_scatter(out_v, [rank], key, mask=m)
                    plsc.store_scatter(out_i, [rank], i * 16 + lane, mask=m)
                    st[4] = optr + jnp.max(ic)
                    st[5] = tptr + jnp.max(ic_t)
        return c
    lax.fori_loop(0, NV // UNROLL, ex, 0)

    pltpu.sync_copy(out_v, topv_hbm.at[pl.ds(wid * PADW, PADW)])
    pltpu.sync_copy(out_i, topi_hbm.at[pl.ds(wid * PADW, PADW)])


def _topk(scores_flat):
    mesh = plsc.VectorSubcoreMesh(core_axis_name="c", subcore_axis_name="s")
    return pl.kernel(
        _topk_kernel,
        out_type=(jax.ShapeDtypeStruct((B * PADW,), jnp.int32),
                  jax.ShapeDtypeStruct((B * PADW,), jnp.int32)),
        mesh=mesh,
        scratch_types=[
            pltpu.VMEM((KN,), jnp.int32),
            pltpu.VMEM((16 * HROW,), jnp.int32),
            pltpu.VMEM((NBINS,), jnp.int32),
            pltpu.VMEM((NBINS,), jnp.int32),
            pltpu.VMEM((PADW,), jnp.int32),
            pltpu.VMEM((PADW,), jnp.int32),
            pltpu.SMEM((8,), jnp.int32),
        ],
        compiler_params=pltpu.CompilerParams(needs_layout_passes=False),
    )(scores_flat)


def _gather_kernel(topi_hbm, know_hbm, out_hbm, idx_v, rows_v, sem):
    wid = lax.axis_index("s") * 2 + lax.axis_index("c")
    pltpu.sync_copy(topi_hbm.at[pl.ds(wid * PADW, PADW)], idx_v)
    pltpu.async_copy(know_hbm.at[idx_v], rows_v, sem).wait()
    pltpu.sync_copy(rows_v, out_hbm.at[pl.ds(wid * PADW, PADW)])


def _gather_rows(topi_flat, knowledge):
    mesh = plsc.VectorSubcoreMesh(core_axis_name="c", subcore_axis_name="s")
    return pl.kernel(
        _gather_kernel,
        out_type=jax.ShapeDtypeStruct((B * PADW, R), jnp.float32),
        mesh=mesh,
        scratch_types=[
            pltpu.VMEM((PADW,), jnp.int32),
            pltpu.VMEM((PADW, R), jnp.float32),
            pltpu.SemaphoreType.DMA,
        ],
        compiler_params=pltpu.CompilerParams(needs_layout_passes=False),
    )(topi_flat, knowledge)


def _finish_body(gath_ref, wq_ref, query_ref, wk_ref, wo_ref,
                 wf1_ref, wf2_ref, vec_ref, out_ref):
    kb = jnp.dot(gath_ref[...], wk_ref[...],
                 preferred_element_type=jnp.float32) + vec_ref[0:1, :]
    kenc = _ln(kb, vec_ref[1:2, :], vec_ref[2:3, :])

    q2 = _ln(jnp.dot(query_ref[...], wq_ref[...],
                     preferred_element_type=jnp.float32) + vec_ref[10:11, :],
             vec_ref[11:12, :], vec_ref[12:13, :])
    sall = jax.lax.dot_general(
        q2, kenc, (((1,), (1,)), ((), ())),
        preferred_element_type=jnp.float32)
    tv = jnp.concatenate(
        [sall[b:b + 1, b * PADW:(b + 1) * PADW] for b in range(B)], axis=0)
    col = lax.broadcasted_iota(jnp.int32, (B, PADW), 1)
    tvm = jnp.where(col < TOPK, tv, -jnp.inf)
    mx = jnp.max(tvm, axis=-1, keepdims=True)
    e = jnp.exp(tvm - mx)
    w = e / jnp.sum(e, axis=-1, keepdims=True)

    wt = jnp.tile(w, (1, B))
    colb = lax.broadcasted_iota(jnp.int32, (B, B * PADW), 1) // PADW
    rowb = lax.broadcasted_iota(jnp.int32, (B, B * PADW), 0)
    w2 = jnp.where(colb == rowb, wt, 0.0)
    ctx = jnp.dot(w2, kenc, preferred_element_type=jnp.float32)

    out = jnp.dot(ctx, wo_ref[...],
                  preferred_element_type=jnp.float32) + vec_ref[3:4, :]
    q = query_ref[...]
    h1 = (jnp.dot(q, wf1_ref[0:H, :], preferred_element_type=jnp.float32)
          + jnp.dot(out, wf1_ref[H:2 * H, :],
                    preferred_element_type=jnp.float32)
          + vec_ref[4:5, :])
    h = jax.nn.gelu(_ln(h1, vec_ref[5:6, :], vec_ref[6:7, :]))
    gate = jax.nn.sigmoid(
        jnp.dot(h, wf2_ref[...], preferred_element_type=jnp.float32)
        + vec_ref[7:8, :])
    out_ref[...] = _ln(q + gate * out, vec_ref[8:9, :], vec_ref[9:10, :])


def _finish(gathered, wq, q2, wk, wo, wf1, wf2, vecs):
    return pl.pallas_call(
        _finish_body,
        in_specs=[
            pl.BlockSpec((B * PADW, R), lambda: (0, 0)),
            pl.BlockSpec((H, R), lambda: (0, 0)),
            pl.BlockSpec((B, H), lambda: (0, 0)),
            pl.BlockSpec((R, R), lambda: (0, 0)),
            pl.BlockSpec((R, H), lambda: (0, 0)),
            pl.BlockSpec((2 * H, H), lambda: (0, 0)),
            pl.BlockSpec((H, H), lambda: (0, 0)),
            pl.BlockSpec((16, H), lambda: (0, 0)),
        ],
        out_specs=pl.BlockSpec((B, H), lambda: (0, 0)),
        out_shape=jax.ShapeDtypeStruct((B, H), jnp.float32),
    )(gathered, wq, q2, wk, wo, wf1, wf2, vecs)


def kernel(query_states, knowledge_embeddings, params):
    p = params
    q2 = query_states.reshape(B, H)
    z = jnp.zeros((H,), jnp.float32)
    vecs1 = jnp.stack([p['bq'], p['gq'], p['betaq'],
                       p['bk'], p['gk'], p['betak'], z, z])
    scores = _encode_score(q2, p['Wq'], p['Wk'], vecs1, knowledge_embeddings)
    topv_flat, topi_flat = _topk(scores.reshape(-1))
    gathered = _gather_rows(topi_flat, knowledge_embeddings)
    vecs2 = jnp.stack([p['bk'], p['gk'], p['betak'], p['bo'], p['bf1'],
                       p['gf'], p['betaf'], p['bf2'], p['gln'], p['bln'],
                       p['bq'], p['gq'], p['betaq'], z, z, z])
    fused = _finish(gathered, p['Wq'], q2,
                    p['Wk'], p['Wo'], p['Wf1'], p['Wf2'], vecs2)
    return fused.reshape(B, 1, H)
